# baseline (speedup 1.0000x reference)
import functools

import jax
import jax.numpy as jnp
from jax import lax
from jax.experimental import pallas as pl
from jax.experimental.pallas import tpu as pltpu
from jax.experimental.pallas import tpu_sc as plsc

NC = 2
NS = 16
NW = NC * NS
L = 16
D = 128
CG = 128


def _mesh():
    return plsc.VectorSubcoreMesh(
        core_axis_name="c", subcore_axis_name="s", num_cores=NC, num_subcores=NS)


_SC_PARAMS = pltpu.CompilerParams(needs_layout_passes=False, use_tc_tiling_on_sc=False)


def _make_deg_kernel(nchunk, npad):
    @functools.partial(
        pl.kernel, mesh=_mesh(), compiler_params=_SC_PARAMS,
        out_type=jax.ShapeDtypeStruct((NW * npad,), jnp.float32),
        scratch_types=[
            pltpu.VMEM((nchunk, CG), jnp.int32),
            pltpu.VMEM((nchunk, CG), jnp.float32),
            pltpu.VMEM((npad,), jnp.float32),
        ])
    def deg_kernel(col_hbm, ew_hbm, degp_hbm, col_v, ew_v, deg_v):
        w = lax.axis_index("s") * NC + lax.axis_index("c")
        pltpu.sync_copy(col_hbm.at[w], col_v)
        pltpu.sync_copy(ew_hbm.at[w], ew_v)

        def zero_body(i, carry):
            deg_v[pl.ds(i * L, L)] = jnp.zeros((L,), jnp.float32)
            return carry
        lax.fori_loop(0, npad // L, zero_body, 0)

        def chunk_body(c, carry):
            def sub(j, carry2):
                c16 = col_v[c, pl.ds(j * L, L)]
                e16 = ew_v[c, pl.ds(j * L, L)]
                plsc.addupdate_scatter(deg_v, [c16], e16)
                return carry2
            return lax.fori_loop(0, CG // L, sub, carry)
        lax.fori_loop(0, nchunk, chunk_body, 0)
        pltpu.sync_copy(deg_v, degp_hbm.at[pl.ds(w * npad, npad)])

    return deg_kernel


def _make_dinv_kernel(npad):
    npt = npad // NW

    @functools.partial(
        pl.kernel, mesh=_mesh(), compiler_params=_SC_PARAMS,
        out_type=jax.ShapeDtypeStruct((npad,), jnp.float32),
        scratch_types=[
            pltpu.VMEM((NW, npt), jnp.float32),
            pltpu.VMEM((npt,), jnp.float32),
        ])
    def dinv_kernel(degp_hbm, dinv_hbm, degs_v, dinv_v):
        w = lax.axis_index("s") * NC + lax.axis_index("c")
        for r in range(NW):
            pltpu.sync_copy(degp_hbm.at[pl.ds(r * npad + w * npt, npt)],
                            degs_v.at[r])

        def body(j, carry):
            acc = jnp.zeros((L,), jnp.float32)
            for r in range(NW):
                acc = acc + degs_v[r, pl.ds(j * L, L)]
            i32 = plsc.bitcast(acc, jnp.int32)
            i32 = jnp.int32(0x5F3759DF) - lax.shift_right_arithmetic(i32, 1)
            y = plsc.bitcast(i32, jnp.float32)
            for _ in range(3):
                y = y * (1.5 - 0.5 * acc * y * y)
            dinv_v[pl.ds(j * L, L)] = y
            return carry
        lax.fori_loop(0, npt // L, body, 0)
        pltpu.sync_copy(dinv_v, dinv_hbm.at[pl.ds(w * npt, npt)])

    return dinv_kernel


def _make_norm_kernel(nchunk, npad, npa):
    @functools.partial(
        pl.kernel, mesh=_mesh(), compiler_params=_SC_PARAMS,
        out_type=jax.ShapeDtypeStruct((NW, nchunk + 3, CG), jnp.float32),
        scratch_types=[
            pltpu.VMEM((nchunk, CG), jnp.int32),
            pltpu.VMEM((nchunk, CG), jnp.int32),
            pltpu.VMEM((nchunk, CG), jnp.float32),
            pltpu.VMEM((npa,), jnp.float32),
            pltpu.VMEM((nchunk, CG), jnp.float32),
        ])
    def norm_kernel(eb_hbm, ew_hbm, dinv_hbm, nrm_hbm,
                    row_v, col_v, ew_v, dinv_v, nrm_v):
        w = lax.axis_index("s") * NC + lax.axis_index("c")
        pltpu.sync_copy(eb_hbm.at[w, 0, pl.ds(0, nchunk)], row_v)
        pltpu.sync_copy(eb_hbm.at[w, 1, pl.ds(0, nchunk)], col_v)
        pltpu.sync_copy(ew_hbm.at[w], ew_v)
        pltpu.sync_copy(dinv_hbm.at[pl.ds(0, npa)], dinv_v)

        def chunk_body(c, carry):
            def sub(j, carry2):
                r16 = row_v[c, pl.ds(j * L, L)]
                c16 = col_v[c, pl.ds(j * L, L)]
                e16 = ew_v[c, pl.ds(j * L, L)]
                dr = plsc.load_gather(dinv_v, [r16])
                dc = plsc.load_gather(dinv_v, [c16])
                nrm_v[c, pl.ds(j * L, L)] = dr * e16 * dc
                return carry2
            return lax.fori_loop(0, CG // L, sub, carry)
        lax.fori_loop(0, nchunk, chunk_body, 0)
        pltpu.sync_copy(nrm_v, nrm_hbm.at[w, pl.ds(0, nchunk)])

    return norm_kernel


def _make_agg_kernel(nchunk, npa):
    spt = npa // NS

    @functools.partial(
        pl.kernel, mesh=_mesh(), compiler_params=_SC_PARAMS,
        out_type=jax.ShapeDtypeStruct((NC, npa, D), jnp.float32),
        scratch_types=[
            pltpu.VMEM((3, CG), jnp.int32),
            pltpu.VMEM((3, CG), jnp.int32),
            pltpu.VMEM((3 * CG,), jnp.float32),
            pltpu.VMEM((3 * CG, D // 2), jnp.int32),
            pltpu.VMEM((CG, D), jnp.float32),
            pltpu.VMEM_SHARED((npa, D), jnp.float32),
            pltpu.SemaphoreType.DMA((3,)),
            pltpu.SemaphoreType.DMA((3,)),
        ])
    def agg_kernel(src_hbm, eb_hbm, nrm_hbm, out_hbm,
                   row_r, col_r, nrm_r, rows_v, scaled_v, accum, esem, gsem):
        cid = lax.axis_index("c")
        sid = lax.axis_index("s")
        w = sid * NC + cid

        def zr(i, carry):
            for j in range(D // L):
                scaled_v[i, pl.ds(j * L, L)] = jnp.zeros((L,), jnp.float32)
            return carry
        lax.fori_loop(0, CG, zr, 0)
        for b in range(spt // CG):
            pltpu.sync_copy(scaled_v,
                            accum.at[pl.ds(sid * spt + b * CG, CG)])
        plsc.subcore_barrier()

        def start_edge(c, b):
            pltpu.async_copy(eb_hbm.at[w, 0, c], row_r.at[b], esem.at[b])
            pltpu.async_copy(eb_hbm.at[w, 1, c], col_r.at[b], esem.at[b])
            pltpu.async_copy(nrm_hbm.at[w, c], nrm_r.at[pl.ds(b * CG, CG)],
                             esem.at[b])

        def wait_edge(b):
            pltpu.make_async_copy(eb_hbm.at[w, 0, 0], row_r.at[b],
                                  esem.at[b]).wait()
            pltpu.make_async_copy(eb_hbm.at[w, 0, 0], col_r.at[b],
                                  esem.at[b]).wait()
            pltpu.make_async_copy(nrm_hbm.at[w, 0],
                                  nrm_r.at[pl.ds(b * CG, CG)],
                                  esem.at[b]).wait()

        def start_gather(b):
            pltpu.async_copy(src_hbm.at[row_r.at[b]],
                             rows_v.at[pl.ds(b * CG, CG)], gsem.at[b])

        def wait_gather(b):
            pltpu.make_async_copy(src_hbm.at[row_r.at[b]],
                                  rows_v.at[pl.ds(b * CG, CG)],
                                  gsem.at[b]).wait()

        start_edge(0, 0)
        start_edge(1, 1)
        start_edge(2, 2)
        wait_edge(0)
        start_gather(0)
        wait_edge(1)
        start_gather(1)

        def step(c, b):
            @pl.when(c + 2 < nchunk)
            def _():
                wait_edge((b + 2) % 3)
                start_gather((b + 2) % 3)

            wait_gather(b)

            def sc_e(p, carry2):
                e = 4 * p
                base = jnp.zeros((L,), jnp.int32) + (b * CG + e)
                nbv = [plsc.load_gather(nrm_r, [base + u]) for u in range(4)]
                for j in range(D // 32):
                    for u in range(4):
                        xi = rows_v[b * CG + e + u, pl.ds(j * L, L)]
                        ae = plsc.bitcast(xi << 16, jnp.float32)
                        ao = plsc.bitcast(xi & jnp.int32(-65536), jnp.float32)
                        scaled_v[e + u, pl.ds(j * 32, L)] = ae * nbv[u]
                        scaled_v[e + u, pl.ds(j * 32 + L, L)] = ao * nbv[u]
                return carry2
            lax.fori_loop(0, CG // 4, sc_e, 0)

            pltpu.sync_copy(scaled_v, accum.at[col_r.at[b]], add=True)

            @pl.when(c + 3 < nchunk)
            def _():
                start_edge(c + 3, b)

        def triple(g, carry):
            step(3 * g, 0)
            step(3 * g + 1, 1)
            step(3 * g + 2, 2)
            return carry
        lax.fori_loop(0, nchunk // 3, triple, 0)
        plsc.subcore_barrier()
        pltpu.sync_copy(accum.at[pl.ds(sid * spt, spt)],
                        out_hbm.at[cid, pl.ds(sid * spt, spt)])

    return agg_kernel


def _relu_mm(p_ref, w_ref, b_ref, o_ref):
    a = p_ref[0] + p_ref[1]
    o_ref[...] = jnp.maximum(
        jnp.dot(a, w_ref[...], preferred_element_type=jnp.float32) + b_ref[...],
        0.0).astype(jnp.bfloat16)


def _mm2(p_ref, w2_ref, b2_ref, w3_ref, b3_ref, m_ref, v_ref):
    a = p_ref[0] + p_ref[1]
    m_ref[...] = jnp.dot(a, w2_ref[...], preferred_element_type=jnp.float32) + b2_ref[...]
    v_ref[...] = jnp.dot(a, w3_ref[...], preferred_element_type=jnp.float32) + b3_ref[...]


def kernel(x, edge_index, edge_weight, W1, b1, W2, b2, W3, b3):
    n, d_in = x.shape
    e = edge_index.shape[1]
    npad = -(-n // 4096) * 4096
    npa = -(-n // (NS * CG)) * NS * CG
    etot = e + n
    nchunk = 3 * (-(-etot // (NW * CG * 3)))
    epad = NW * CG * nchunk

    loop = jnp.arange(n, dtype=jnp.int32)
    row = jnp.concatenate([edge_index[0], loop])
    col = jnp.concatenate([edge_index[1], loop])
    ew = jnp.concatenate([edge_weight, jnp.ones((n,), jnp.float32)])
    pad = epad - etot
    row3 = jnp.pad(row, (0, pad)).reshape(NW, nchunk, CG)
    col3 = jnp.pad(col, (0, pad)).reshape(NW, nchunk, CG)
    ew3 = jnp.pad(ew, (0, pad)).reshape(NW, nchunk, CG)
    eb = jnp.stack([row3, col3], axis=1)
    eb = jnp.pad(eb, ((0, 0), (0, 0), (0, 3), (0, 0)))
    degp = _make_deg_kernel(nchunk, npad)(col3, ew3)
    dinv = _make_dinv_kernel(npad)(degp)
    nrm = _make_norm_kernel(nchunk, npad, npa)(eb, ew3, dinv)

    g = jnp.arange(D // 32)[:, None]
    i = jnp.arange(16)[None, :]
    perm = jnp.stack([g * 32 + i, g * 32 + 16 + i], axis=2).reshape(-1)
    xb = x[:, perm].astype(jnp.bfloat16).view(jnp.int32)
    W1p = W1[:, perm]
    b1p = b1[perm]

    agg = _make_agg_kernel(nchunk, npa)
    p1 = agg(xb, eb, nrm)

    bn = 512
    h = pl.pallas_call(
        _relu_mm,
        grid=(npa // bn,),
        in_specs=[
            pl.BlockSpec((NC, bn, D), lambda i: (0, i, 0)),
            pl.BlockSpec((D, D), lambda i: (0, 0)),
            pl.BlockSpec((1, D), lambda i: (0, 0)),
        ],
        out_specs=pl.BlockSpec((bn, D), lambda i: (i, 0)),
        out_shape=jax.ShapeDtypeStruct((npa, D), jnp.bfloat16),
    )(p1, W1p, b1p.reshape(1, D))

    p2 = agg(h.view(jnp.int32), eb, nrm)

    mean, var = pl.pallas_call(
        _mm2,
        grid=(npa // bn,),
        in_specs=[
            pl.BlockSpec((NC, bn, D), lambda i: (0, i, 0)),
            pl.BlockSpec((D, D), lambda i: (0, 0)),
            pl.BlockSpec((1, D), lambda i: (0, 0)),
            pl.BlockSpec((D, D), lambda i: (0, 0)),
            pl.BlockSpec((1, D), lambda i: (0, 0)),
        ],
        out_specs=[
            pl.BlockSpec((bn, D), lambda i: (i, 0)),
            pl.BlockSpec((bn, D), lambda i: (i, 0)),
        ],
        out_shape=[
            jax.ShapeDtypeStruct((npa, D), jnp.float32),
            jax.ShapeDtypeStruct((npa, D), jnp.float32),
        ],
    )(p2, W2, b2.reshape(1, D), W3, b3.reshape(1, D))

    return (mean[:n], var[:n])


# speedup vs baseline: 1.2572x; 1.2572x over previous
import functools

import jax
import jax.numpy as jnp
from jax import lax
from jax.experimental import pallas as pl
from jax.experimental.pallas import tpu as pltpu
from jax.experimental.pallas import tpu_sc as plsc

NC = 2
NS = 16
NW = NC * NS
L = 16
D = 128
CG = 128


def _mesh():
    return plsc.VectorSubcoreMesh(
        core_axis_name="c", subcore_axis_name="s", num_cores=NC, num_subcores=NS)


_SC_PARAMS = pltpu.CompilerParams(needs_layout_passes=False, use_tc_tiling_on_sc=False)


def _make_deg_kernel(nchunk, npad):
    @functools.partial(
        pl.kernel, mesh=_mesh(), compiler_params=_SC_PARAMS,
        out_type=jax.ShapeDtypeStruct((NW * npad,), jnp.float32),
        scratch_types=[
            pltpu.VMEM((nchunk, CG), jnp.int32),
            pltpu.VMEM((nchunk, CG), jnp.float32),
            pltpu.VMEM((npad,), jnp.float32),
        ])
    def deg_kernel(col_hbm, ew_hbm, degp_hbm, col_v, ew_v, deg_v):
        w = lax.axis_index("s") * NC + lax.axis_index("c")
        pltpu.sync_copy(col_hbm.at[w], col_v)
        pltpu.sync_copy(ew_hbm.at[w], ew_v)

        def zero_body(i, carry):
            deg_v[pl.ds(i * L, L)] = jnp.zeros((L,), jnp.float32)
            return carry
        lax.fori_loop(0, npad // L, zero_body, 0)

        def chunk_body(c, carry):
            def sub(j, carry2):
                c16 = col_v[c, pl.ds(j * L, L)]
                e16 = ew_v[c, pl.ds(j * L, L)]
                plsc.addupdate_scatter(deg_v, [c16], e16)
                return carry2
            return lax.fori_loop(0, CG // L, sub, carry)
        lax.fori_loop(0, nchunk, chunk_body, 0)
        pltpu.sync_copy(deg_v, degp_hbm.at[pl.ds(w * npad, npad)])

    return deg_kernel


def _make_dinv_kernel(npad):
    npt = npad // NW

    @functools.partial(
        pl.kernel, mesh=_mesh(), compiler_params=_SC_PARAMS,
        out_type=jax.ShapeDtypeStruct((npad,), jnp.float32),
        scratch_types=[
            pltpu.VMEM((NW, npt), jnp.float32),
            pltpu.VMEM((npt,), jnp.float32),
        ])
    def dinv_kernel(degp_hbm, dinv_hbm, degs_v, dinv_v):
        w = lax.axis_index("s") * NC + lax.axis_index("c")
        for r in range(NW):
            pltpu.sync_copy(degp_hbm.at[pl.ds(r * npad + w * npt, npt)],
                            degs_v.at[r])

        def body(j, carry):
            acc = jnp.zeros((L,), jnp.float32)
            for r in range(NW):
                acc = acc + degs_v[r, pl.ds(j * L, L)]
            i32 = plsc.bitcast(acc, jnp.int32)
            i32 = jnp.int32(0x5F3759DF) - lax.shift_right_arithmetic(i32, 1)
            y = plsc.bitcast(i32, jnp.float32)
            for _ in range(3):
                y = y * (1.5 - 0.5 * acc * y * y)
            dinv_v[pl.ds(j * L, L)] = y
            return carry
        lax.fori_loop(0, npt // L, body, 0)
        pltpu.sync_copy(dinv_v, dinv_hbm.at[pl.ds(w * npt, npt)])

    return dinv_kernel


def _make_norm_kernel(nchunk, npad, npa):
    @functools.partial(
        pl.kernel, mesh=_mesh(), compiler_params=_SC_PARAMS,
        out_type=jax.ShapeDtypeStruct((NW, nchunk + 3, CG), jnp.float32),
        scratch_types=[
            pltpu.VMEM((nchunk, CG), jnp.int32),
            pltpu.VMEM((nchunk, CG), jnp.int32),
            pltpu.VMEM((nchunk, CG), jnp.float32),
            pltpu.VMEM((npa,), jnp.float32),
            pltpu.VMEM((nchunk, CG), jnp.float32),
        ])
    def norm_kernel(eb_hbm, ew_hbm, dinv_hbm, nrm_hbm,
                    row_v, col_v, ew_v, dinv_v, nrm_v):
        w = lax.axis_index("s") * NC + lax.axis_index("c")
        pltpu.sync_copy(eb_hbm.at[w, 0, pl.ds(0, nchunk)], row_v)
        pltpu.sync_copy(eb_hbm.at[w, 1, pl.ds(0, nchunk)], col_v)
        pltpu.sync_copy(ew_hbm.at[w], ew_v)
        pltpu.sync_copy(dinv_hbm.at[pl.ds(0, npa)], dinv_v)

        def chunk_body(c, carry):
            def sub(j, carry2):
                r16 = row_v[c, pl.ds(j * L, L)]
                c16 = col_v[c, pl.ds(j * L, L)]
                e16 = ew_v[c, pl.ds(j * L, L)]
                dr = plsc.load_gather(dinv_v, [r16])
                dc = plsc.load_gather(dinv_v, [c16])
                nrm_v[c, pl.ds(j * L, L)] = dr * e16 * dc
                return carry2
            return lax.fori_loop(0, CG // L, sub, carry)
        lax.fori_loop(0, nchunk, chunk_body, 0)
        pltpu.sync_copy(nrm_v, nrm_hbm.at[w, pl.ds(0, nchunk)])

    return norm_kernel


def _make_agg_kernel(nchunk, npa):
    spt = npa // NS

    @functools.partial(
        pl.kernel, mesh=_mesh(), compiler_params=_SC_PARAMS,
        out_type=jax.ShapeDtypeStruct((NC, npa, D), jnp.float32),
        scratch_types=[
            pltpu.VMEM((3, CG), jnp.int32),
            pltpu.VMEM((3, CG), jnp.int32),
            pltpu.VMEM((3 * CG,), jnp.float32),
            pltpu.VMEM((3 * CG, D // 2), jnp.int32),
            pltpu.VMEM((CG, D), jnp.float32),
            pltpu.VMEM_SHARED((npa, D), jnp.float32),
            pltpu.SemaphoreType.DMA((3,)),
            pltpu.SemaphoreType.DMA((3,)),
        ])
    def agg_kernel(src_hbm, eb_hbm, nrm_hbm, out_hbm,
                   row_r, col_r, nrm_r, rows_v, scaled_v, accum, esem, gsem):
        cid = lax.axis_index("c")
        sid = lax.axis_index("s")
        w = sid * NC + cid

        def zr(i, carry):
            for j in range(D // L):
                scaled_v[i, pl.ds(j * L, L)] = jnp.zeros((L,), jnp.float32)
            return carry
        lax.fori_loop(0, CG, zr, 0)
        for b in range(spt // CG):
            pltpu.sync_copy(scaled_v,
                            accum.at[pl.ds(sid * spt + b * CG, CG)])
        plsc.subcore_barrier()

        def start_edge(c, b):
            pltpu.async_copy(eb_hbm.at[w, 0, c], row_r.at[b], esem.at[b])
            pltpu.async_copy(eb_hbm.at[w, 1, c], col_r.at[b], esem.at[b])
            pltpu.async_copy(nrm_hbm.at[w, c], nrm_r.at[pl.ds(b * CG, CG)],
                             esem.at[b])

        def wait_edge(b):
            pltpu.make_async_copy(eb_hbm.at[w, 0, 0], row_r.at[b],
                                  esem.at[b]).wait()
            pltpu.make_async_copy(eb_hbm.at[w, 0, 0], col_r.at[b],
                                  esem.at[b]).wait()
            pltpu.make_async_copy(nrm_hbm.at[w, 0],
                                  nrm_r.at[pl.ds(b * CG, CG)],
                                  esem.at[b]).wait()

        def start_gather(b):
            pltpu.async_copy(src_hbm.at[row_r.at[b]],
                             rows_v.at[pl.ds(b * CG, CG)], gsem.at[b])

        def wait_gather(b):
            pltpu.make_async_copy(src_hbm.at[row_r.at[b]],
                                  rows_v.at[pl.ds(b * CG, CG)],
                                  gsem.at[b]).wait()

        start_edge(0, 0)
        start_edge(1, 1)
        start_edge(2, 2)
        wait_edge(0)
        start_gather(0)
        wait_edge(1)
        start_gather(1)

        def step(c, b):
            @pl.when(c + 2 < nchunk)
            def _():
                wait_edge((b + 2) % 3)
                start_gather((b + 2) % 3)

            wait_gather(b)

            def sc_e(p, carry2):
                e = 2 * p
                base = jnp.zeros((L,), jnp.int32) + (b * CG + e)
                nbv0 = plsc.load_gather(nrm_r, [base])
                nbv1 = plsc.load_gather(nrm_r, [base + 1])
                for j in range(D // 32):
                    xi0 = rows_v[b * CG + e, pl.ds(j * L, L)]
                    xi1 = rows_v[b * CG + e + 1, pl.ds(j * L, L)]
                    ae0 = plsc.bitcast(xi0 << 16, jnp.float32)
                    ao0 = plsc.bitcast(xi0 & jnp.int32(-65536), jnp.float32)
                    ae1 = plsc.bitcast(xi1 << 16, jnp.float32)
                    ao1 = plsc.bitcast(xi1 & jnp.int32(-65536), jnp.float32)
                    scaled_v[e, pl.ds(j * 32, L)] = ae0 * nbv0
                    scaled_v[e, pl.ds(j * 32 + L, L)] = ao0 * nbv0
                    scaled_v[e + 1, pl.ds(j * 32, L)] = ae1 * nbv1
                    scaled_v[e + 1, pl.ds(j * 32 + L, L)] = ao1 * nbv1
                return carry2
            lax.fori_loop(0, CG // 2, sc_e, 0)

            pltpu.sync_copy(scaled_v, accum.at[col_r.at[b]], add=True)

            @pl.when(c + 3 < nchunk)
            def _():
                start_edge(c + 3, b)

        def triple(g, carry):
            step(3 * g, 0)
            step(3 * g + 1, 1)
            step(3 * g + 2, 2)
            return carry
        lax.fori_loop(0, nchunk // 3, triple, 0)
        plsc.subcore_barrier()
        pltpu.sync_copy(accum.at[pl.ds(sid * spt, spt)],
                        out_hbm.at[cid, pl.ds(sid * spt, spt)])

    return agg_kernel


def _relu_mm(p_ref, w_ref, b_ref, o_ref):
    a = p_ref[0] + p_ref[1]
    o_ref[...] = jnp.maximum(
        jnp.dot(a, w_ref[...], preferred_element_type=jnp.float32) + b_ref[...],
        0.0).astype(jnp.bfloat16)


def _mm2(p_ref, w2_ref, b2_ref, w3_ref, b3_ref, m_ref, v_ref):
    a = p_ref[0] + p_ref[1]
    m_ref[...] = jnp.dot(a, w2_ref[...], preferred_element_type=jnp.float32) + b2_ref[...]
    v_ref[...] = jnp.dot(a, w3_ref[...], preferred_element_type=jnp.float32) + b3_ref[...]


def kernel(x, edge_index, edge_weight, W1, b1, W2, b2, W3, b3):
    n, d_in = x.shape
    e = edge_index.shape[1]
    npad = -(-n // 4096) * 4096
    npa = -(-n // (NS * CG)) * NS * CG
    etot = e + n
    nchunk = 3 * (-(-etot // (NW * CG * 3)))
    epad = NW * CG * nchunk

    loop = jnp.arange(n, dtype=jnp.int32)
    row = jnp.concatenate([edge_index[0], loop])
    col = jnp.concatenate([edge_index[1], loop])
    ew = jnp.concatenate([edge_weight, jnp.ones((n,), jnp.float32)])
    pad = epad - etot
    row3 = jnp.pad(row, (0, pad)).reshape(NW, nchunk, CG)
    col3 = jnp.pad(col, (0, pad)).reshape(NW, nchunk, CG)
    ew3 = jnp.pad(ew, (0, pad)).reshape(NW, nchunk, CG)
    eb = jnp.stack([row3, col3], axis=1)
    eb = jnp.pad(eb, ((0, 0), (0, 0), (0, 3), (0, 0)))
    degp = _make_deg_kernel(nchunk, npad)(col3, ew3)
    dinv = _make_dinv_kernel(npad)(degp)
    nrm = _make_norm_kernel(nchunk, npad, npa)(eb, ew3, dinv)

    g = jnp.arange(D // 32)[:, None]
    i = jnp.arange(16)[None, :]
    perm = jnp.stack([g * 32 + i, g * 32 + 16 + i], axis=2).reshape(-1)
    xb = x[:, perm].astype(jnp.bfloat16).view(jnp.int32)
    W1p = W1[:, perm]
    b1p = b1[perm]

    agg = _make_agg_kernel(nchunk, npa)
    p1 = agg(xb, eb, nrm)

    bn = 512
    h = pl.pallas_call(
        _relu_mm,
        grid=(npa // bn,),
        in_specs=[
            pl.BlockSpec((NC, bn, D), lambda i: (0, i, 0)),
            pl.BlockSpec((D, D), lambda i: (0, 0)),
            pl.BlockSpec((1, D), lambda i: (0, 0)),
        ],
        out_specs=pl.BlockSpec((bn, D), lambda i: (i, 0)),
        out_shape=jax.ShapeDtypeStruct((npa, D), jnp.bfloat16),
    )(p1, W1p, b1p.reshape(1, D))

    p2 = agg(h.view(jnp.int32), eb, nrm)

    mean, var = pl.pallas_call(
        _mm2,
        grid=(npa // bn,),
        in_specs=[
            pl.BlockSpec((NC, bn, D), lambda i: (0, i, 0)),
            pl.BlockSpec((D, D), lambda i: (0, 0)),
            pl.BlockSpec((1, D), lambda i: (0, 0)),
            pl.BlockSpec((D, D), lambda i: (0, 0)),
            pl.BlockSpec((1, D), lambda i: (0, 0)),
        ],
        out_specs=[
            pl.BlockSpec((bn, D), lambda i: (i, 0)),
            pl.BlockSpec((bn, D), lambda i: (i, 0)),
        ],
        out_shape=[
            jax.ShapeDtypeStruct((npa, D), jnp.float32),
            jax.ShapeDtypeStruct((npa, D), jnp.float32),
        ],
    )(p2, W2, b2.reshape(1, D), W3, b3.reshape(1, D))

    return (mean[:n], var[:n])
